# baseline (speedup 1.0000x reference)
import functools

import jax
import jax.numpy as jnp
from jax import lax
from jax.experimental import pallas as pl
from jax.experimental.pallas import tpu as pltpu
from jax.experimental.pallas import tpu_sc as plsc


def _fused_table_body(emb_ref, w_ref, b_ref, out_ref):
    out_ref[0] = (
        jnp.dot(emb_ref[...], w_ref[...], preferred_element_type=jnp.float32)
        + b_ref[...]
    )


def _make_fused_table(vocab, d_out, copies):
    return pl.pallas_call(
        _fused_table_body,
        grid=(copies,),
        in_specs=[
            pl.BlockSpec((vocab, 16), lambda i: (0, 0)),
            pl.BlockSpec((16, d_out), lambda i: (0, 0)),
            pl.BlockSpec((1, d_out), lambda i: (0, 0)),
        ],
        out_specs=pl.BlockSpec((1, vocab, d_out), lambda i: (i, 0, 0)),
        out_shape=jax.ShapeDtypeStruct((copies, vocab, d_out), jnp.float32),
    )


def _make_gather(n_tokens, d_out, chunk):
    info = plsc.get_sparse_core_info()
    nw = info.num_cores * info.num_subcores
    per_w = n_tokens // nw
    n_chunks = per_w // chunk
    assert n_tokens % nw == 0 and per_w % chunk == 0
    assert chunk <= 128 and per_w % 8 == 0 and chunk % 8 == 0
    assert n_chunks >= 10 and (n_chunks - 8) % 4 == 0

    mesh = plsc.VectorSubcoreMesh(core_axis_name="c", subcore_axis_name="s")

    @functools.partial(
        pl.kernel,
        mesh=mesh,
        out_type=jax.ShapeDtypeStruct((n_tokens, d_out), jnp.float32),
        scratch_types=[
            pltpu.VMEM((4, chunk), jnp.int32),
            pltpu.VMEM((4, chunk, d_out), jnp.float32),
        ] + [pltpu.SemaphoreType.DMA] * 8,
    )
    def gather(table_hbm, idx_hbm, out_hbm, idx_v, rows_v, *sems):
        wid = lax.axis_index("s") * info.num_cores + lax.axis_index("c")
        base = wid * per_w
        gsem = sems[:4]
        osem = sems[4:]

        def idx_load(j, b):
            pltpu.sync_copy(idx_hbm.at[pl.ds(base + j * chunk, chunk)],
                            idx_v.at[b])

        def gather_copy(b):
            return pltpu.make_async_copy(table_hbm.at[idx_v.at[b]],
                                         rows_v.at[b], gsem[b])

        def out_copy(j, b):
            return pltpu.make_async_copy(
                rows_v.at[b], out_hbm.at[pl.ds(base + j * chunk, chunk)],
                osem[b])

        def step(j, b, has_idx, has_wwait, has_g):
            gather_copy(b).wait()
            out_copy(j, b).start()
            if has_idx:
                idx_load(j + 4, b)
            b2 = (b + 2) % 4
            if has_wwait:
                out_copy(j - 2, b2).wait()
            if has_g:
                gather_copy(b2).start()

        for b in range(4):
            idx_load(b, b)
        for b in range(2):
            gather_copy(b).start()

        for j in range(2):
            step(j, j, True, False, True)

        def outer(g, carry):
            for bb in range(4):
                j = 4 * g + 2 + bb
                step(j, (2 + bb) % 4, True, True, True)
            return carry

        lax.fori_loop(0, (n_chunks - 8) // 4, outer, 0)

        for t in range(6):
            j = n_chunks - 6 + t
            step(j, j % 4, j + 4 < n_chunks, True, j + 2 < n_chunks)
        for j in (n_chunks - 2, n_chunks - 1):
            out_copy(j, j % 4).wait()

    return gather


def kernel(input_ids, emb_table, W, b):
    batch, seqlen = input_ids.shape
    vocab, d_in = emb_table.shape
    d_out = W.shape[1]
    n_tokens = batch * seqlen

    info = plsc.get_sparse_core_info()
    nw = info.num_cores * info.num_subcores
    per_w = n_tokens // nw

    fused = _make_fused_table(vocab, d_out, nw)(
        emb_table, W, b.reshape(1, d_out)
    )
    ids = input_ids.astype(jnp.int32).T.reshape(n_tokens)
    ids = ids + (jnp.arange(n_tokens, dtype=jnp.int32) // per_w) * vocab
    out = _make_gather(n_tokens, d_out, 80)(
        fused.reshape(nw * vocab, d_out), ids
    )
    return out.reshape(seqlen, batch, d_out).transpose(1, 0, 2)


# speedup vs baseline: 1.0182x; 1.0182x over previous
import functools

import jax
import jax.numpy as jnp
from jax import lax
from jax.experimental import pallas as pl
from jax.experimental.pallas import tpu as pltpu
from jax.experimental.pallas import tpu_sc as plsc


def _fused_table_body(emb_ref, w_ref, b_ref, out_ref):
    out_ref[0] = (
        jnp.dot(emb_ref[...], w_ref[...], preferred_element_type=jnp.float32)
        + b_ref[...]
    )


def _make_fused_table(vocab, d_out, copies):
    return pl.pallas_call(
        _fused_table_body,
        grid=(copies,),
        in_specs=[
            pl.BlockSpec((vocab, 16), lambda i: (0, 0)),
            pl.BlockSpec((16, d_out), lambda i: (0, 0)),
            pl.BlockSpec((1, d_out), lambda i: (0, 0)),
        ],
        out_specs=pl.BlockSpec((1, vocab, d_out), lambda i: (i, 0, 0)),
        out_shape=jax.ShapeDtypeStruct((copies, vocab, d_out), jnp.float32),
    )


def _make_gather(n_tokens, d_out, chunk):
    info = plsc.get_sparse_core_info()
    nw = info.num_cores * info.num_subcores
    per_w = n_tokens // nw
    n_chunks = per_w // chunk
    assert n_tokens % nw == 0 and per_w % chunk == 0
    assert chunk <= 128 and per_w % 8 == 0 and (chunk * nw) % 8 == 0
    assert n_chunks >= 4 and n_chunks % 2 == 0

    mesh = plsc.VectorSubcoreMesh(core_axis_name="c", subcore_axis_name="s")

    @functools.partial(
        pl.kernel,
        mesh=mesh,
        out_type=jax.ShapeDtypeStruct((n_tokens, d_out), jnp.float32),
        scratch_types=[
            pltpu.VMEM((2, chunk), jnp.int32),
            pltpu.VMEM((2, chunk, d_out), jnp.float32),
            pltpu.SemaphoreType.DMA,
            pltpu.SemaphoreType.DMA,
            pltpu.SemaphoreType.DMA,
            pltpu.SemaphoreType.DMA,
        ],
    )
    def gather(table_hbm, idx_hbm, out_hbm, idx_v, rows_v, g0, g1, o0, o1):
        wid = lax.axis_index("s") * info.num_cores + lax.axis_index("c")
        base = wid * per_w
        gsem = [g0, g1]
        osem = [o0, o1]

        def idx_load(j, b):
            pltpu.sync_copy(idx_hbm.at[pl.ds(base + j * chunk, chunk)],
                            idx_v.at[b])

        def gather_copy(b, sem):
            return pltpu.make_async_copy(table_hbm.at[idx_v.at[b]],
                                         rows_v.at[b], sem)

        def out_copy(j, b, sem):
            return pltpu.make_async_copy(
                rows_v.at[b], out_hbm.at[pl.ds(base + j * chunk, chunk)], sem)

        for b in range(2):
            idx_load(b, b)
            gather_copy(b, gsem[b]).start()

        def outer(g, carry):
            for b in range(2):
                j = 2 * g + b
                gather_copy(b, gsem[b]).wait()
                out_copy(j, b, osem[b]).start()
                idx_load(j + 2, b)
                out_copy(j, b, osem[b]).wait()
                gather_copy(b, gsem[b]).start()
            return carry

        lax.fori_loop(0, (n_chunks - 2) // 2, outer, 0)

        for b in range(2):
            j = n_chunks - 2 + b
            gather_copy(b, gsem[b]).wait()
            out_copy(j, b, osem[b]).start()
        for b in range(2):
            j = n_chunks - 2 + b
            out_copy(j, b, osem[b]).wait()

    return gather


def kernel(input_ids, emb_table, W, b):
    batch, seqlen = input_ids.shape
    vocab, d_in = emb_table.shape
    d_out = W.shape[1]
    n_tokens = batch * seqlen

    info = plsc.get_sparse_core_info()
    nw = info.num_cores * info.num_subcores
    per_w = n_tokens // nw

    fused = _make_fused_table(vocab, d_out, nw)(
        emb_table, W, b.reshape(1, d_out)
    )
    ids = input_ids.astype(jnp.int32).T.reshape(n_tokens)
    ids = ids + (jnp.arange(n_tokens, dtype=jnp.int32) // per_w) * vocab
    out = _make_gather(n_tokens, d_out, 128)(
        fused.reshape(nw * vocab, d_out), ids
    )
    return out.reshape(seqlen, batch, d_out).transpose(1, 0, 2)


# speedup vs baseline: 1.0188x; 1.0006x over previous
import functools

import jax
import jax.numpy as jnp
from jax import lax
from jax.experimental import pallas as pl
from jax.experimental.pallas import tpu as pltpu
from jax.experimental.pallas import tpu_sc as plsc


def _fused_table_body(emb_ref, w_ref, b_ref, out_ref):
    out_ref[0] = (
        jnp.dot(emb_ref[...], w_ref[...], preferred_element_type=jnp.float32)
        + b_ref[...]
    )


def _make_fused_table(vocab, d_out, copies):
    return pl.pallas_call(
        _fused_table_body,
        grid=(copies,),
        in_specs=[
            pl.BlockSpec((vocab, 16), lambda i: (0, 0)),
            pl.BlockSpec((16, d_out), lambda i: (0, 0)),
            pl.BlockSpec((1, d_out), lambda i: (0, 0)),
        ],
        out_specs=pl.BlockSpec((1, vocab, d_out), lambda i: (i, 0, 0)),
        out_shape=jax.ShapeDtypeStruct((copies, vocab, d_out), jnp.float32),
    )


def _make_gather(n_tokens, d_out, chunk):
    info = plsc.get_sparse_core_info()
    nw = info.num_cores * info.num_subcores
    per_w = n_tokens // nw
    n_chunks = per_w // chunk
    D = 4
    assert n_tokens % nw == 0 and per_w % chunk == 0
    assert chunk <= 128 and (chunk * nw) % 8 == 0
    assert n_chunks >= 2 * D and n_chunks % D == 0

    mesh = plsc.VectorSubcoreMesh(core_axis_name="c", subcore_axis_name="s")

    @functools.partial(
        pl.kernel,
        mesh=mesh,
        out_type=jax.ShapeDtypeStruct((n_tokens, d_out), jnp.float32),
        scratch_types=[
            pltpu.VMEM((D, chunk), jnp.int32),
            pltpu.VMEM((D, chunk, d_out), jnp.float32),
        ] + [pltpu.SemaphoreType.DMA] * (2 * D),
    )
    def gather(table_hbm, idx_hbm, out_hbm, idx_v, rows_v,
               g0, g1, g2, g3, o0, o1, o2, o3):
        wid = lax.axis_index("s") * info.num_cores + lax.axis_index("c")
        base = wid * per_w
        gsem = [g0, g1, g2, g3]
        osem = [o0, o1, o2, o3]

        def idx_load(j, b):
            pltpu.sync_copy(idx_hbm.at[pl.ds(base + j * chunk, chunk)],
                            idx_v.at[b])

        def gather_copy(b):
            return pltpu.make_async_copy(table_hbm.at[idx_v.at[b]],
                                         rows_v.at[b], gsem[b])

        def out_copy(j, b):
            return pltpu.make_async_copy(
                rows_v.at[b], out_hbm.at[pl.ds(base + j * chunk, chunk)],
                osem[b])

        def step(j, b, bg, wait_out, start_gather):
            if wait_out:
                out_copy(j - 2, bg).wait()
            if start_gather:
                idx_load(j + 2, bg)
                gather_copy(bg).start()
            gather_copy(b).wait()
            out_copy(j, b).start()

        for b in range(2):
            idx_load(b, b)
            gather_copy(b).start()

        for j in range(2):
            step(j, j, j + 2, False, True)

        def outer(g, carry):
            for u in range(D):
                step(D * g + 2 + u, (2 + u) % D, u, True, True)
            return carry

        lax.fori_loop(0, (n_chunks - D) // D, outer, 0)

        for j in range(n_chunks - 2, n_chunks):
            step(j, j % D, (j - 2) % D, True, False)
        for j in range(n_chunks - 2, n_chunks):
            out_copy(j, j % D).wait()

    return gather


def kernel(input_ids, emb_table, W, b):
    batch, seqlen = input_ids.shape
    vocab, d_in = emb_table.shape
    d_out = W.shape[1]
    n_tokens = batch * seqlen

    info = plsc.get_sparse_core_info()
    nw = info.num_cores * info.num_subcores
    per_w = n_tokens // nw

    fused = _make_fused_table(vocab, d_out, nw)(
        emb_table, W, b.reshape(1, d_out)
    )
    ids = input_ids.astype(jnp.int32).T.reshape(n_tokens)
    ids = ids + (jnp.arange(n_tokens, dtype=jnp.int32) // per_w) * vocab
    out = _make_gather(n_tokens, d_out, 64)(
        fused.reshape(nw * vocab, d_out), ids
    )
    return out.reshape(seqlen, batch, d_out).transpose(1, 0, 2)


# speedup vs baseline: 1.0772x; 1.0573x over previous
import functools

import jax
import jax.numpy as jnp
from jax import lax
from jax.experimental import pallas as pl
from jax.experimental.pallas import tpu as pltpu
from jax.experimental.pallas import tpu_sc as plsc


def _fused_table_body(emb_ref, w_ref, b_ref, out_ref):
    t = (
        jnp.dot(emb_ref[...], w_ref[...], preferred_element_type=jnp.float32)
        + b_ref[...]
    )
    out_ref[...] = jnp.broadcast_to(t, out_ref.shape)


def _make_fused_table(vocab, d_out, copies):
    return pl.pallas_call(
        _fused_table_body,
        out_shape=jax.ShapeDtypeStruct((copies, vocab, d_out), jnp.float32),
    )


def _make_gather(n_tokens, d_out, chunk):
    info = plsc.get_sparse_core_info()
    nw = info.num_cores * info.num_subcores
    per_w = n_tokens // nw
    n_chunks = per_w // chunk
    D = 4
    assert n_tokens % nw == 0 and per_w % chunk == 0
    assert chunk <= 128 and (chunk * nw) % 8 == 0
    assert n_chunks >= 2 * D and n_chunks % D == 0

    mesh = plsc.VectorSubcoreMesh(core_axis_name="c", subcore_axis_name="s")

    @functools.partial(
        pl.kernel,
        mesh=mesh,
        out_type=jax.ShapeDtypeStruct((n_tokens, d_out), jnp.float32),
        scratch_types=[
            pltpu.VMEM((D, chunk), jnp.int32),
            pltpu.VMEM((D, chunk, d_out), jnp.float32),
        ] + [pltpu.SemaphoreType.DMA] * (2 * D),
    )
    def gather(table_hbm, idx_hbm, out_hbm, idx_v, rows_v,
               g0, g1, g2, g3, o0, o1, o2, o3):
        wid = lax.axis_index("s") * info.num_cores + lax.axis_index("c")
        base = wid * per_w
        gsem = [g0, g1, g2, g3]
        osem = [o0, o1, o2, o3]

        def idx_load(j, b):
            pltpu.sync_copy(idx_hbm.at[pl.ds(base + j * chunk, chunk)],
                            idx_v.at[b])

        def gather_copy(b):
            return pltpu.make_async_copy(table_hbm.at[idx_v.at[b]],
                                         rows_v.at[b], gsem[b])

        def out_copy(j, b):
            return pltpu.make_async_copy(
                rows_v.at[b], out_hbm.at[pl.ds(base + j * chunk, chunk)],
                osem[b])

        def step(j, b, bg, wait_out, start_gather):
            if wait_out:
                out_copy(j - 2, bg).wait()
            if start_gather:
                idx_load(j + 2, bg)
                gather_copy(bg).start()
            gather_copy(b).wait()
            out_copy(j, b).start()

        for b in range(2):
            idx_load(b, b)
            gather_copy(b).start()

        for j in range(2):
            step(j, j, j + 2, False, True)

        def outer(g, carry):
            for u in range(D):
                step(D * g + 2 + u, (2 + u) % D, u, True, True)
            return carry

        lax.fori_loop(0, (n_chunks - D) // D, outer, 0)

        for j in range(n_chunks - 2, n_chunks):
            step(j, j % D, (j - 2) % D, True, False)
        for j in range(n_chunks - 2, n_chunks):
            out_copy(j, j % D).wait()

    return gather


def kernel(input_ids, emb_table, W, b):
    batch, seqlen = input_ids.shape
    vocab, d_in = emb_table.shape
    d_out = W.shape[1]
    n_tokens = batch * seqlen

    info = plsc.get_sparse_core_info()
    nw = info.num_cores * info.num_subcores
    per_w = n_tokens // nw

    fused = _make_fused_table(vocab, d_out, nw)(
        emb_table, W, b.reshape(1, d_out)
    )
    ids = input_ids.astype(jnp.int32).T.reshape(n_tokens)
    ids = ids + (jnp.arange(n_tokens, dtype=jnp.int32) // per_w) * vocab
    out = _make_gather(n_tokens, d_out, 64)(
        fused.reshape(nw * vocab, d_out), ids
    )
    return out.reshape(seqlen, batch, d_out).transpose(1, 0, 2)
